# baseline (speedup 1.0000x reference)
import functools
import numpy as np
import jax
import jax.numpy as jnp
from jax import lax
from jax.experimental import pallas as pl
from jax.experimental.pallas import tpu as pltpu
from jax.experimental.pallas import tpu_sc as plsc

_N_LEVELS = 16
_T = 2 ** 19
_BASE_RES = 16
_SCALE = 1.4472692012786865
_N_HASH = 4
_MASK = _T - 1
_P1 = -1640531535
_P2 = 805459861

_RES = [int(np.floor(_BASE_RES * _SCALE ** l)) for l in range(_N_LEVELS)]
_N_DENSE = sum(1 for r in _RES if (r + 1) ** 3 <= _T)

_B = 131072
_NC, _NS = 2, 16
_NW = _NC * _NS
_BW = _B // _NW
_CH = 512
_NSUB = _BW // _CH
_NG = _CH // 16
_NIDX = _CH * 8
_IDX_ROWS = _NIDX // 128


def _sc_body(tbl_hbm, tab4_hbm, x_hbm, code_hbm, resf_hbm, stridef_hbm, out_hbm,
             x_st, code_st, idx0, idx1, rows0, rows1, wc0, wc1,
             out_cv, lvl_stage, resf_s, stride_s, sem0, sem1):
    wid = lax.axis_index("s") * _NC + lax.axis_index("c")
    iota = lax.iota(jnp.int32, 16)
    iota3 = iota * 3
    iota4 = iota * 4

    pltpu.sync_copy(resf_hbm, lvl_stage)
    rv = lvl_stage[pl.ds(0, 16)]
    pltpu.sync_copy(stridef_hbm, lvl_stage)
    sv = lvl_stage[pl.ds(0, 16)]
    for l in range(_N_LEVELS):
        resf_s[l] = rv[l]
        stride_s[l] = sv[l]

    idx_bufs = (idx0, idx1)
    rows_bufs = (rows0, rows1)
    wc_bufs = (wc0, wc1)
    sems = (sem0, sem1)

    def pass1(l, par):
        idx_buf = idx_bufs[par]
        wc_v = wc_bufs[par]
        resf = resf_s[l]
        stridef = stride_s[l]
        stride = stridef.astype(jnp.int32)
        s2 = stride * stride
        lbase = l * _T
        is_dense = l < _N_DENSE

        def p1(g, carry):
            x0 = plsc.load_gather(x_st, [iota3 + g * 48])
            x1 = plsc.load_gather(x_st, [iota3 + (g * 48 + 1)])
            x2 = plsc.load_gather(x_st, [iota3 + (g * 48 + 2)])
            p0 = (x0 * resf).astype(jnp.int32)
            p1i = (x1 * resf).astype(jnp.int32)
            p2 = (x2 * resf).astype(jnp.int32)
            w0 = x0 * resf - p0.astype(jnp.float32)
            w1 = x1 * resf - p1i.astype(jnp.float32)
            w2 = x2 * resf - p2.astype(jnp.float32)
            m0 = 1.0 - w0
            m1 = 1.0 - w1
            m2 = 1.0 - w2
            a0 = p0
            b0 = p0 + 1
            a1 = p1i * _P1
            b1 = a1 + _P1
            a2 = p2 * _P2
            b2 = a2 + _P2
            dbase = p0 + p1i * stride + p2 * s2 + lbase
            col = (g & 7) * 16
            row0 = g >> 3
            for c in range(8):
                o0, o1, o2 = c & 1, (c >> 1) & 1, (c >> 2) & 1
                h = (b0 if o0 else a0) ^ (b1 if o1 else a1) ^ (b2 if o2 else a2)
                idx_h = (h & _MASK) + lbase
                idx_d = dbase + (o0 + stride * o1 + s2 * o2)
                idx = jnp.where(is_dense, idx_d, idx_h)
                wc = ((w0 if o0 else m0) * (w1 if o1 else m1)) * (w2 if o2 else m2)
                idx_buf[4 * c + row0, pl.ds(col, 16)] = idx
                wc_v[pl.ds(c * _CH + g * 16, 16)] = wc
            return carry
        lax.fori_loop(0, _NG, p1, 0)

    def fire(par):
        idx_buf = idx_bufs[par]
        rows_v = rows_bufs[par]
        sem = sems[par]

        def f(k, carry):
            pltpu.make_async_copy(
                tbl_hbm.at[idx_buf.at[k]],
                rows_v.at[pl.ds(k * 128, 128)], sem).start()
            return carry
        lax.fori_loop(0, _IDX_ROWS, f, 0)

    def drain(par):
        idx_buf = idx_bufs[par]
        rows_v = rows_bufs[par]
        sem = sems[par]

        def f(k, carry):
            pltpu.make_async_copy(
                tbl_hbm.at[idx_buf.at[k]],
                rows_v.at[pl.ds(k * 128, 128)], sem).wait()
            return carry
        lax.fori_loop(0, _IDX_ROWS, f, 0)

    def pass2(l, par):
        rows_v = rows_bufs[par]
        wc_v = wc_bufs[par]

        def p2(g, carry):
            cc0 = plsc.load_gather(code_st, [iota4 + g * 64])
            cc1 = plsc.load_gather(code_st, [iota4 + (g * 64 + 1)])
            cc2 = plsc.load_gather(code_st, [iota4 + (g * 64 + 2)])
            cc3 = plsc.load_gather(code_st, [iota4 + (g * 64 + 3)])
            f0 = jnp.zeros((16,), jnp.float32)
            f1 = jnp.zeros((16,), jnp.float32)
            for c in range(8):
                wc = wc_v[pl.ds(c * _CH + g * 16, 16)]
                coef = (wc * cc0, wc * cc1, wc * cc2, wc * cc3)
                d0 = c * _CH + g * 16 + iota
                for j in range(8):
                    d2 = jnp.full((16,), j, jnp.int32)
                    v = plsc.load_gather(rows_v, [d0, d2])
                    if j & 1:
                        f1 = f1 + coef[j >> 1] * v
                    else:
                        f0 = f0 + coef[j >> 1] * v
            rowv = g * 16 + iota
            plsc.store_scatter(
                out_cv, [rowv, jnp.full((16,), 2 * l, jnp.int32)], f0)
            plsc.store_scatter(
                out_cv, [rowv, jnp.full((16,), 2 * l + 1, jnp.int32)], f1)
            return carry
        lax.fori_loop(0, _NG, p2, 0)

    def sub_chunk(s, carry):
        base = wid * _BW + s * _CH
        pltpu.sync_copy(x_hbm.at[pl.ds(base * 3, 3 * _CH)], x_st)
        pltpu.sync_copy(code_hbm.at[pl.ds(base * 4, 4 * _CH)], code_st)

        pass1(0, 0)
        fire(0)

        def pair(i, c):
            l = 2 * i
            pass1(l + 1, 1)
            fire(1)
            drain(0)
            pass2(l, 0)

            @pl.when(l + 2 < _N_LEVELS)
            def _():
                pass1(l + 2, 0)
                fire(0)
            drain(1)
            pass2(l + 1, 1)
            return c
        lax.fori_loop(0, _N_LEVELS // 2, pair, 0)

        pltpu.sync_copy(out_cv, out_hbm.at[pl.ds(base, _CH), :])
        return carry

    lax.fori_loop(0, _NSUB, sub_chunk, 0)


_TPB = 2048
_TPW = _T // _NW
_TPBLKS = _TPW // _TPB


def _tp_body(tflat_hbm, tbl8_hbm, st, ov):
    wid = lax.axis_index("s") * _NC + lax.axis_index("c")
    iota = lax.iota(jnp.int32, 16)
    i_half = iota >> 1
    i_par = iota & 1

    def level(l, carry):
        def blk(b, c2):
            r0 = wid * _TPW + b * _TPB
            for h in range(_N_HASH):
                pltpu.sync_copy(
                    tflat_hbm.at[pl.ds((h * _N_LEVELS + l) * 2 * _T + 2 * r0,
                                       2 * _TPB)],
                    st.at[h])

            def q_loop(q, c3):
                dr = i_half + q * 8
                for h in range(_N_HASH):
                    v = st[h, pl.ds(q * 16, 16)]
                    dc = i_par + 2 * h
                    plsc.store_scatter(ov, [dr, dc], v)
                return c3
            lax.fori_loop(0, _TPB // 8, q_loop, 0)
            pltpu.sync_copy(ov, tbl8_hbm.at[pl.ds(l * _T + r0, _TPB), :])
            return c2
        lax.fori_loop(0, _TPBLKS, blk, 0)
        return carry
    lax.fori_loop(0, _N_LEVELS, level, 0)


_tp_call = functools.partial(
    pl.kernel,
    mesh=plsc.VectorSubcoreMesh(core_axis_name="c", subcore_axis_name="s"),
    compiler_params=pltpu.CompilerParams(
        needs_layout_passes=False, use_tc_tiling_on_sc=False),
    out_type=jax.ShapeDtypeStruct((_N_LEVELS * _T, _N_HASH * 2), jnp.float32),
    scratch_types=[
        pltpu.VMEM((_N_HASH, 2 * _TPB), jnp.float32),
        pltpu.VMEM((_TPB, _N_HASH * 2), jnp.float32),
    ],
)(_tp_body)


_sc_call = functools.partial(
    pl.kernel,
    mesh=plsc.VectorSubcoreMesh(core_axis_name="c", subcore_axis_name="s"),
    compiler_params=pltpu.CompilerParams(
        needs_layout_passes=False, use_tc_tiling_on_sc=False),
    out_type=jax.ShapeDtypeStruct((_B, 2 * _N_LEVELS), jnp.float32),
    scratch_types=[
        pltpu.VMEM((3 * _CH,), jnp.float32),
        pltpu.VMEM((4 * _CH,), jnp.float32),
        pltpu.VMEM((_IDX_ROWS, 128), jnp.int32),
        pltpu.VMEM((_IDX_ROWS, 128), jnp.int32),
        pltpu.VMEM((_NIDX, 8), jnp.float32),
        pltpu.VMEM((_NIDX, 8), jnp.float32),
        pltpu.VMEM((_NIDX,), jnp.float32),
        pltpu.VMEM((_NIDX,), jnp.float32),
        pltpu.VMEM((_CH, 2 * _N_LEVELS), jnp.float32),
        pltpu.VMEM((_N_LEVELS,), jnp.float32),
        pltpu.SMEM((_N_LEVELS,), jnp.float32),
        pltpu.SMEM((_N_LEVELS,), jnp.float32),
        pltpu.SemaphoreType.DMA,
        pltpu.SemaphoreType.DMA,
    ],
)(_sc_body)


def kernel(in_tensor, conditioning_code, tables):
    tbl8 = jnp.concatenate(
        [tables[h].reshape(_N_LEVELS * _T, 2) for h in range(_N_HASH)], axis=1)
    resf = jnp.asarray([float(r) for r in _RES], jnp.float32)
    stridef = jnp.asarray([float(r + 1) for r in _RES], jnp.float32)
    return _sc_call(tbl8, tables, in_tensor.reshape(-1),
                    conditioning_code.reshape(-1), resf, stridef)


# speedup vs baseline: 23.3091x; 23.3091x over previous
import functools
import numpy as np
import jax
import jax.numpy as jnp
from jax import lax
from jax.experimental import pallas as pl
from jax.experimental.pallas import tpu as pltpu
from jax.experimental.pallas import tpu_sc as plsc

_N_LEVELS = 16
_T = 2 ** 19
_BASE_RES = 16
_SCALE = 1.4472692012786865
_N_HASH = 4
_MASK = _T - 1
_P1 = -1640531535
_P2 = 805459861

_RES = [int(np.floor(_BASE_RES * _SCALE ** l)) for l in range(_N_LEVELS)]
_N_DENSE = sum(1 for r in _RES if (r + 1) ** 3 <= _T)

_B = 131072
_NC, _NS = 2, 16
_NW = _NC * _NS
_BW = _B // _NW
_CH = 512
_NSUB = _BW // _CH
_NG = _CH // 16
_NIDX = _CH * 8
_IDX_ROWS = _NIDX // 128


def _sc_body(tbl_hbm, x_hbm, code_hbm, resf_hbm, stridef_hbm, out_hbm,
             x_st, code_st, idx0, idx1, rows0, rows1, wc0, wc1,
             out_cv, lvl_stage, resf_s, stride_s, sem0, sem1):
    wid = lax.axis_index("s") * _NC + lax.axis_index("c")
    iota = lax.iota(jnp.int32, 16)
    iota3 = iota * 3
    iota4 = iota * 4

    pltpu.sync_copy(resf_hbm, lvl_stage)
    rv = lvl_stage[pl.ds(0, 16)]
    pltpu.sync_copy(stridef_hbm, lvl_stage)
    sv = lvl_stage[pl.ds(0, 16)]
    for l in range(_N_LEVELS):
        resf_s[l] = rv[l]
        stride_s[l] = sv[l]

    idx_bufs = (idx0, idx1)
    rows_bufs = (rows0, rows1)
    wc_bufs = (wc0, wc1)
    sems = (sem0, sem1)

    def pass1(l, par):
        idx_buf = idx_bufs[par]
        wc_v = wc_bufs[par]
        resf = resf_s[l]
        stridef = stride_s[l]
        stride = stridef.astype(jnp.int32)
        s2 = stride * stride
        lbase = l * _T
        is_dense = l < _N_DENSE

        def p1(g, carry):
            x0 = plsc.load_gather(x_st, [iota3 + g * 48])
            x1 = plsc.load_gather(x_st, [iota3 + (g * 48 + 1)])
            x2 = plsc.load_gather(x_st, [iota3 + (g * 48 + 2)])
            p0 = (x0 * resf).astype(jnp.int32)
            p1i = (x1 * resf).astype(jnp.int32)
            p2 = (x2 * resf).astype(jnp.int32)
            w0 = x0 * resf - p0.astype(jnp.float32)
            w1 = x1 * resf - p1i.astype(jnp.float32)
            w2 = x2 * resf - p2.astype(jnp.float32)
            m0 = 1.0 - w0
            m1 = 1.0 - w1
            m2 = 1.0 - w2
            a0 = p0
            b0 = p0 + 1
            a1 = p1i * _P1
            b1 = a1 + _P1
            a2 = p2 * _P2
            b2 = a2 + _P2
            dbase = p0 + p1i * stride + p2 * s2 + lbase
            col = (g & 7) * 16
            row0 = g >> 3
            for c in range(8):
                o0, o1, o2 = c & 1, (c >> 1) & 1, (c >> 2) & 1
                h = (b0 if o0 else a0) ^ (b1 if o1 else a1) ^ (b2 if o2 else a2)
                idx_h = (h & _MASK) + lbase
                idx_d = dbase + (o0 + stride * o1 + s2 * o2)
                idx = jnp.where(is_dense, idx_d, idx_h)
                wc = ((w0 if o0 else m0) * (w1 if o1 else m1)) * (w2 if o2 else m2)
                idx_buf[4 * c + row0, pl.ds(col, 16)] = idx
                wc_v[pl.ds(c * _CH + g * 16, 16)] = wc
            return carry
        lax.fori_loop(0, _NG, p1, 0)

    def fire(par):
        idx_buf = idx_bufs[par]
        rows_v = rows_bufs[par]
        sem = sems[par]

        def f(k, carry):
            pltpu.make_async_copy(
                tbl_hbm.at[idx_buf.at[k]],
                rows_v.at[pl.ds(k * 128, 128)], sem).start()
            return carry
        lax.fori_loop(0, _IDX_ROWS, f, 0)

    def drain(par):
        idx_buf = idx_bufs[par]
        rows_v = rows_bufs[par]
        sem = sems[par]

        def f(k, carry):
            pltpu.make_async_copy(
                tbl_hbm.at[idx_buf.at[k]],
                rows_v.at[pl.ds(k * 128, 128)], sem).wait()
            return carry
        lax.fori_loop(0, _IDX_ROWS, f, 0)

    def pass2(l, par):
        rows_v = rows_bufs[par]
        wc_v = wc_bufs[par]

        def p2(g, carry):
            cc0 = plsc.load_gather(code_st, [iota4 + g * 64])
            cc1 = plsc.load_gather(code_st, [iota4 + (g * 64 + 1)])
            cc2 = plsc.load_gather(code_st, [iota4 + (g * 64 + 2)])
            cc3 = plsc.load_gather(code_st, [iota4 + (g * 64 + 3)])
            f0 = jnp.zeros((16,), jnp.float32)
            f1 = jnp.zeros((16,), jnp.float32)
            for c in range(8):
                wc = wc_v[pl.ds(c * _CH + g * 16, 16)]
                coef = (wc * cc0, wc * cc1, wc * cc2, wc * cc3)
                d0 = c * _CH + g * 16 + iota
                for j in range(8):
                    d2 = jnp.full((16,), j, jnp.int32)
                    v = plsc.load_gather(rows_v, [d0, d2])
                    if j & 1:
                        f1 = f1 + coef[j >> 1] * v
                    else:
                        f0 = f0 + coef[j >> 1] * v
            rowv = g * 16 + iota
            plsc.store_scatter(
                out_cv, [rowv, jnp.full((16,), 2 * l, jnp.int32)], f0)
            plsc.store_scatter(
                out_cv, [rowv, jnp.full((16,), 2 * l + 1, jnp.int32)], f1)
            return carry
        lax.fori_loop(0, _NG, p2, 0)

    def sub_chunk(s, carry):
        base = wid * _BW + s * _CH
        pltpu.sync_copy(x_hbm.at[pl.ds(base * 3, 3 * _CH)], x_st)
        pltpu.sync_copy(code_hbm.at[pl.ds(base * 4, 4 * _CH)], code_st)

        pass1(0, 0)
        fire(0)

        def pair(i, c):
            l = 2 * i
            pass1(l + 1, 1)
            fire(1)
            drain(0)
            pass2(l, 0)

            @pl.when(l + 2 < _N_LEVELS)
            def _():
                pass1(l + 2, 0)
                fire(0)
            drain(1)
            pass2(l + 1, 1)
            return c
        lax.fori_loop(0, _N_LEVELS // 2, pair, 0)

        pltpu.sync_copy(out_cv, out_hbm.at[pl.ds(base, _CH), :])
        return carry

    lax.fori_loop(0, _NSUB, sub_chunk, 0)


_TPB = 2048
_TPW = _T // _NW
_TPBLKS = _TPW // _TPB


def _tp_body(t4_hbm, tbl8_hbm, st, ov):
    wid = lax.axis_index("s") * _NC + lax.axis_index("c")
    iota = lax.iota(jnp.int32, 16)
    i_top = iota >> 3
    i_par = iota & 1
    i_col = iota & 7
    dh = (iota & 7) >> 1

    def level(l, carry):
        def blk(b, c2):
            r0 = wid * _TPW + b * _TPB
            for h in range(_N_HASH):
                pltpu.sync_copy(
                    t4_hbm.at[h, l, pl.ds(r0 // 64, _TPB // 64), :],
                    st.at[h])

            def q_loop(q, c3):
                t = i_top + 2 * q
                dr6 = t >> 6
                dcol = ((t & 63) << 1) + i_par
                v = plsc.load_gather(st, [dh, dr6, dcol])
                plsc.store_scatter(ov, [t, i_col], v)
                return c3
            lax.fori_loop(0, _TPB // 2, q_loop, 0)
            pltpu.sync_copy(ov, tbl8_hbm.at[pl.ds(l * _T + r0, _TPB), :])
            return c2
        lax.fori_loop(0, _TPBLKS, blk, 0)
        return carry
    lax.fori_loop(0, _N_LEVELS, level, 0)


_tp_call = functools.partial(
    pl.kernel,
    mesh=plsc.VectorSubcoreMesh(core_axis_name="c", subcore_axis_name="s"),
    compiler_params=pltpu.CompilerParams(
        needs_layout_passes=False, use_tc_tiling_on_sc=False),
    out_type=jax.ShapeDtypeStruct((_N_LEVELS * _T, _N_HASH * 2), jnp.float32),
    scratch_types=[
        pltpu.VMEM((_N_HASH, _TPB // 64, 128), jnp.float32),
        pltpu.VMEM((_TPB, _N_HASH * 2), jnp.float32),
    ],
)(_tp_body)


_sc_call = functools.partial(
    pl.kernel,
    mesh=plsc.VectorSubcoreMesh(core_axis_name="c", subcore_axis_name="s"),
    compiler_params=pltpu.CompilerParams(
        needs_layout_passes=False, use_tc_tiling_on_sc=False),
    out_type=jax.ShapeDtypeStruct((_B, 2 * _N_LEVELS), jnp.float32),
    scratch_types=[
        pltpu.VMEM((3 * _CH,), jnp.float32),
        pltpu.VMEM((4 * _CH,), jnp.float32),
        pltpu.VMEM((_IDX_ROWS, 128), jnp.int32),
        pltpu.VMEM((_IDX_ROWS, 128), jnp.int32),
        pltpu.VMEM((_NIDX, 8), jnp.float32),
        pltpu.VMEM((_NIDX, 8), jnp.float32),
        pltpu.VMEM((_NIDX,), jnp.float32),
        pltpu.VMEM((_NIDX,), jnp.float32),
        pltpu.VMEM((_CH, 2 * _N_LEVELS), jnp.float32),
        pltpu.VMEM((_N_LEVELS,), jnp.float32),
        pltpu.SMEM((_N_LEVELS,), jnp.float32),
        pltpu.SMEM((_N_LEVELS,), jnp.float32),
        pltpu.SemaphoreType.DMA,
        pltpu.SemaphoreType.DMA,
    ],
)(_sc_body)


def kernel(in_tensor, conditioning_code, tables):
    tbl8 = _tp_call(tables.reshape(_N_HASH, _N_LEVELS, _T // 64, 128))
    resf = jnp.asarray([float(r) for r in _RES], jnp.float32)
    stridef = jnp.asarray([float(r + 1) for r in _RES], jnp.float32)
    return _sc_call(tbl8, in_tensor.reshape(-1),
                    conditioning_code.reshape(-1), resf, stridef)


# speedup vs baseline: 27.0997x; 1.1626x over previous
import functools
import numpy as np
import jax
import jax.numpy as jnp
from jax import lax
from jax.experimental import pallas as pl
from jax.experimental.pallas import tpu as pltpu
from jax.experimental.pallas import tpu_sc as plsc

_N_LEVELS = 16
_T = 2 ** 19
_BASE_RES = 16
_SCALE = 1.4472692012786865
_N_HASH = 4
_MASK = _T - 1
_P1 = -1640531535
_P2 = 805459861

_RES = [int(np.floor(_BASE_RES * _SCALE ** l)) for l in range(_N_LEVELS)]
_N_DENSE = sum(1 for r in _RES if (r + 1) ** 3 <= _T)

_B = 131072
_NC, _NS = 2, 16
_NW = _NC * _NS
_BW = _B // _NW
_CH = 512
_NSUB = _BW // _CH
_NG = _CH // 16
_NIDX = _CH * 8
_IDX_ROWS = _NIDX // 128


def _sc_body(tbl_hbm, x_hbm, code_hbm, resf_hbm, stridef_hbm, out_hbm,
             x_st, code_st, idx0, idx1, rows0, rows1, wc0, wc1,
             out_cv, lvl_stage, resf_s, stride_s, sem0, sem1):
    wid = lax.axis_index("s") * _NC + lax.axis_index("c")
    iota = lax.iota(jnp.int32, 16)
    iota3 = iota * 3
    iota4 = iota * 4

    pltpu.sync_copy(resf_hbm, lvl_stage)
    rv = lvl_stage[pl.ds(0, 16)]
    pltpu.sync_copy(stridef_hbm, lvl_stage)
    sv = lvl_stage[pl.ds(0, 16)]
    for l in range(_N_LEVELS):
        resf_s[l] = rv[l]
        stride_s[l] = sv[l]

    idx_bufs = (idx0, idx1)
    rows_bufs = (rows0, rows1)
    wc_bufs = (wc0, wc1)
    sems = (sem0, sem1)

    def pass1(l, par):
        idx_buf = idx_bufs[par]
        wc_v = wc_bufs[par]
        resf = resf_s[l]
        stridef = stride_s[l]
        stride = stridef.astype(jnp.int32)
        s2 = stride * stride
        lbase = l * _T
        is_dense = l < _N_DENSE

        def p1(g, carry):
            f0i = iota3 + g * 48
            f1i = iota3 + (g * 48 + 1)
            f2i = iota3 + (g * 48 + 2)
            x0 = plsc.load_gather(x_st, [f0i >> 7, f0i & 127])
            x1 = plsc.load_gather(x_st, [f1i >> 7, f1i & 127])
            x2 = plsc.load_gather(x_st, [f2i >> 7, f2i & 127])
            p0 = (x0 * resf).astype(jnp.int32)
            p1i = (x1 * resf).astype(jnp.int32)
            p2 = (x2 * resf).astype(jnp.int32)
            w0 = x0 * resf - p0.astype(jnp.float32)
            w1 = x1 * resf - p1i.astype(jnp.float32)
            w2 = x2 * resf - p2.astype(jnp.float32)
            m0 = 1.0 - w0
            m1 = 1.0 - w1
            m2 = 1.0 - w2
            a0 = p0
            b0 = p0 + 1
            a1 = p1i * _P1
            b1 = a1 + _P1
            a2 = p2 * _P2
            b2 = a2 + _P2
            dbase = p0 + p1i * stride + p2 * s2 + lbase
            col = (g & 7) * 16
            row0 = g >> 3
            for c in range(8):
                o0, o1, o2 = c & 1, (c >> 1) & 1, (c >> 2) & 1
                h = (b0 if o0 else a0) ^ (b1 if o1 else a1) ^ (b2 if o2 else a2)
                idx_h = (h & _MASK) + lbase
                idx_d = dbase + (o0 + stride * o1 + s2 * o2)
                idx = jnp.where(is_dense, idx_d, idx_h)
                wc = ((w0 if o0 else m0) * (w1 if o1 else m1)) * (w2 if o2 else m2)
                idx_buf[4 * c + row0, pl.ds(col, 16)] = idx
                wc_v[pl.ds(c * _CH + g * 16, 16)] = wc
            return carry
        lax.fori_loop(0, _NG, p1, 0)

    def fire(par):
        idx_buf = idx_bufs[par]
        rows_v = rows_bufs[par]
        sem = sems[par]

        def f(k, carry):
            pltpu.make_async_copy(
                tbl_hbm.at[idx_buf.at[k]],
                rows_v.at[pl.ds(k * 128, 128)], sem).start()
            return carry
        lax.fori_loop(0, _IDX_ROWS, f, 0)

    def drain(par):
        idx_buf = idx_bufs[par]
        rows_v = rows_bufs[par]
        sem = sems[par]

        def f(k, carry):
            pltpu.make_async_copy(
                tbl_hbm.at[idx_buf.at[k]],
                rows_v.at[pl.ds(k * 128, 128)], sem).wait()
            return carry
        lax.fori_loop(0, _IDX_ROWS, f, 0)

    def pass2(l, par):
        rows_v = rows_bufs[par]
        wc_v = wc_bufs[par]

        def p2(g, carry):
            g0i = iota4 + g * 64
            g1i = iota4 + (g * 64 + 1)
            g2i = iota4 + (g * 64 + 2)
            g3i = iota4 + (g * 64 + 3)
            cc0 = plsc.load_gather(code_st, [g0i >> 7, g0i & 127])
            cc1 = plsc.load_gather(code_st, [g1i >> 7, g1i & 127])
            cc2 = plsc.load_gather(code_st, [g2i >> 7, g2i & 127])
            cc3 = plsc.load_gather(code_st, [g3i >> 7, g3i & 127])
            f0 = jnp.zeros((16,), jnp.float32)
            f1 = jnp.zeros((16,), jnp.float32)
            for c in range(8):
                wc = wc_v[pl.ds(c * _CH + g * 16, 16)]
                coef = (wc * cc0, wc * cc1, wc * cc2, wc * cc3)
                d0 = c * _CH + g * 16 + iota
                for j in range(8):
                    d2 = jnp.full((16,), j, jnp.int32)
                    v = plsc.load_gather(rows_v, [d0, d2])
                    if j & 1:
                        f1 = f1 + coef[j >> 1] * v
                    else:
                        f0 = f0 + coef[j >> 1] * v
            rowv = g * 16 + iota
            plsc.store_scatter(
                out_cv, [rowv, jnp.full((16,), 2 * l, jnp.int32)], f0)
            plsc.store_scatter(
                out_cv, [rowv, jnp.full((16,), 2 * l + 1, jnp.int32)], f1)
            return carry
        lax.fori_loop(0, _NG, p2, 0)

    def sub_chunk(s, carry):
        base = wid * _BW + s * _CH
        pltpu.sync_copy(x_hbm.at[pl.ds(wid * 96 + s * 12, 12), :], x_st)
        pltpu.sync_copy(code_hbm.at[pl.ds(wid * 128 + s * 16, 16), :], code_st)

        pass1(0, 0)
        fire(0)

        def pair(i, c):
            l = 2 * i
            pass1(l + 1, 1)
            fire(1)
            drain(0)
            pass2(l, 0)

            @pl.when(l + 2 < _N_LEVELS)
            def _():
                pass1(l + 2, 0)
                fire(0)
            drain(1)
            pass2(l + 1, 1)
            return c
        lax.fori_loop(0, _N_LEVELS // 2, pair, 0)

        pltpu.sync_copy(out_cv, out_hbm.at[pl.ds(base, _CH), :])
        return carry

    lax.fori_loop(0, _NSUB, sub_chunk, 0)


_TPB = 4096
_TPW = _T // _NW
_TPBLKS = _TPW // _TPB


def _tp_body(t4_hbm, tbl8_hbm, st, ov):
    wid = lax.axis_index("s") * _NC + lax.axis_index("c")
    iota = lax.iota(jnp.int32, 16)
    i_half = iota >> 1
    i_par = iota & 1

    def level(l, carry):
        def blk(b, c2):
            r0 = wid * _TPW + b * _TPB
            for h in range(_N_HASH):
                pltpu.sync_copy(
                    t4_hbm.at[h, l, pl.ds(r0 // 64, _TPB // 64), :],
                    st.at[h])

            def q_loop(q, c3):
                dr = i_half + q * 8
                for h in range(_N_HASH):
                    v = st[h, q >> 3, pl.ds((q & 7) * 16, 16)]
                    plsc.store_scatter(ov, [dr, i_par + 2 * h], v)
                return c3
            lax.fori_loop(0, _TPB // 8, q_loop, 0)
            pltpu.sync_copy(ov, tbl8_hbm.at[pl.ds(l * _T + r0, _TPB), :])
            return c2
        lax.fori_loop(0, _TPBLKS, blk, 0)
        return carry
    lax.fori_loop(0, _N_LEVELS, level, 0)


_tp_call = functools.partial(
    pl.kernel,
    mesh=plsc.VectorSubcoreMesh(core_axis_name="c", subcore_axis_name="s"),
    compiler_params=pltpu.CompilerParams(
        needs_layout_passes=False, use_tc_tiling_on_sc=False),
    out_type=jax.ShapeDtypeStruct((_N_LEVELS * _T, _N_HASH * 2), jnp.float32),
    scratch_types=[
        pltpu.VMEM((_N_HASH, _TPB // 64, 128), jnp.float32),
        pltpu.VMEM((_TPB, _N_HASH * 2), jnp.float32),
    ],
)(_tp_body)


_sc_call = functools.partial(
    pl.kernel,
    mesh=plsc.VectorSubcoreMesh(core_axis_name="c", subcore_axis_name="s"),
    compiler_params=pltpu.CompilerParams(
        needs_layout_passes=False, use_tc_tiling_on_sc=False),
    out_type=jax.ShapeDtypeStruct((_B, 2 * _N_LEVELS), jnp.float32),
    scratch_types=[
        pltpu.VMEM((12, 128), jnp.float32),
        pltpu.VMEM((16, 128), jnp.float32),
        pltpu.VMEM((_IDX_ROWS, 128), jnp.int32),
        pltpu.VMEM((_IDX_ROWS, 128), jnp.int32),
        pltpu.VMEM((_NIDX, 8), jnp.float32),
        pltpu.VMEM((_NIDX, 8), jnp.float32),
        pltpu.VMEM((_NIDX,), jnp.float32),
        pltpu.VMEM((_NIDX,), jnp.float32),
        pltpu.VMEM((_CH, 2 * _N_LEVELS), jnp.float32),
        pltpu.VMEM((_N_LEVELS,), jnp.float32),
        pltpu.SMEM((_N_LEVELS,), jnp.float32),
        pltpu.SMEM((_N_LEVELS,), jnp.float32),
        pltpu.SemaphoreType.DMA,
        pltpu.SemaphoreType.DMA,
    ],
)(_sc_body)


def kernel(in_tensor, conditioning_code, tables):
    tbl8 = _tp_call(tables.reshape(_N_HASH, _N_LEVELS, _T // 64, 128))
    resf = jnp.asarray([float(r) for r in _RES], jnp.float32)
    stridef = jnp.asarray([float(r + 1) for r in _RES], jnp.float32)
    return _sc_call(tbl8, in_tensor.reshape(_B * 3 // 128, 128),
                    conditioning_code.reshape(_B * 4 // 128, 128),
                    resf, stridef)


# speedup vs baseline: 32.4261x; 1.1965x over previous
import functools
import numpy as np
import jax
import jax.numpy as jnp
from jax import lax
from jax.experimental import pallas as pl
from jax.experimental.pallas import tpu as pltpu
from jax.experimental.pallas import tpu_sc as plsc

_N_LEVELS = 16
_T = 2 ** 19
_BASE_RES = 16
_SCALE = 1.4472692012786865
_N_HASH = 4
_MASK = _T - 1
_P1 = -1640531535
_P2 = 805459861

_RES = [int(np.floor(_BASE_RES * _SCALE ** l)) for l in range(_N_LEVELS)]
_N_DENSE = sum(1 for r in _RES if (r + 1) ** 3 <= _T)

_B = 131072
_NC, _NS = 2, 16
_NW = _NC * _NS
_BW = _B // _NW
_CH = 512
_NSUB = _BW // _CH
_NG = _CH // 16
_NIDX = _CH * 8
_IDX_ROWS = _NIDX // 128


def _sc_body(tbl_hbm, x_hbm, code_hbm, resf_hbm, stridef_hbm, out_hbm,
             x_st, code_st, idx0, idx1, rows0, rows1, wc0, wc1,
             out_cv, lvl_stage, resf_s, stride_s, sem0, sem1):
    wid = lax.axis_index("s") * _NC + lax.axis_index("c")
    iota = lax.iota(jnp.int32, 16)
    iota3 = iota * 3
    iota4 = iota * 4

    pltpu.sync_copy(resf_hbm, lvl_stage)
    rv = lvl_stage[pl.ds(0, 16)]
    pltpu.sync_copy(stridef_hbm, lvl_stage)
    sv = lvl_stage[pl.ds(0, 16)]
    for l in range(_N_LEVELS):
        resf_s[l] = rv[l]
        stride_s[l] = sv[l]

    idx_bufs = (idx0, idx1)
    rows_bufs = (rows0, rows1)
    wc_bufs = (wc0, wc1)
    sems = (sem0, sem1)

    def pass1(l, par):
        idx_buf = idx_bufs[par]
        wc_v = wc_bufs[par]
        resf = resf_s[l]
        stridef = stride_s[l]
        stride = stridef.astype(jnp.int32)
        s2 = stride * stride
        lbase = l * _T
        is_dense = l < _N_DENSE

        def p1(g, carry):
            f0i = iota3 + g * 48
            f1i = iota3 + (g * 48 + 1)
            f2i = iota3 + (g * 48 + 2)
            x0 = plsc.load_gather(x_st, [f0i >> 7, f0i & 127])
            x1 = plsc.load_gather(x_st, [f1i >> 7, f1i & 127])
            x2 = plsc.load_gather(x_st, [f2i >> 7, f2i & 127])
            p0 = (x0 * resf).astype(jnp.int32)
            p1i = (x1 * resf).astype(jnp.int32)
            p2 = (x2 * resf).astype(jnp.int32)
            w0 = x0 * resf - p0.astype(jnp.float32)
            w1 = x1 * resf - p1i.astype(jnp.float32)
            w2 = x2 * resf - p2.astype(jnp.float32)
            m0 = 1.0 - w0
            m1 = 1.0 - w1
            m2 = 1.0 - w2
            a0 = p0
            b0 = p0 + 1
            a1 = p1i * _P1
            b1 = a1 + _P1
            a2 = p2 * _P2
            b2 = a2 + _P2
            dbase = p0 + p1i * stride + p2 * s2 + lbase
            col = (g & 7) * 16
            row0 = g >> 3
            for c in range(8):
                o0, o1, o2 = c & 1, (c >> 1) & 1, (c >> 2) & 1
                h = (b0 if o0 else a0) ^ (b1 if o1 else a1) ^ (b2 if o2 else a2)
                idx_h = (h & _MASK) + lbase
                idx_d = dbase + (o0 + stride * o1 + s2 * o2)
                idx = jnp.where(is_dense, idx_d, idx_h)
                wc = ((w0 if o0 else m0) * (w1 if o1 else m1)) * (w2 if o2 else m2)
                idx_buf[4 * c + row0, pl.ds(col, 16)] = idx
                wc_v[pl.ds(c * _CH + g * 16, 16)] = wc
            return carry
        lax.fori_loop(0, _NG, p1, 0)

    def fire(par):
        idx_buf = idx_bufs[par]
        rows_v = rows_bufs[par]
        sem = sems[par]

        def f(k, carry):
            pltpu.make_async_copy(
                tbl_hbm.at[idx_buf.at[k]],
                rows_v.at[pl.ds(k * 128, 128)], sem).start()
            return carry
        lax.fori_loop(0, _IDX_ROWS, f, 0)

    def drain(par):
        idx_buf = idx_bufs[par]
        rows_v = rows_bufs[par]
        sem = sems[par]

        def f(k, carry):
            pltpu.make_async_copy(
                tbl_hbm.at[idx_buf.at[k]],
                rows_v.at[pl.ds(k * 128, 128)], sem).wait()
            return carry
        lax.fori_loop(0, _IDX_ROWS, f, 0)

    def pass2(l, par):
        rows_v = rows_bufs[par]
        wc_v = wc_bufs[par]

        def p2(g, carry):
            g0i = iota4 + g * 64
            g1i = iota4 + (g * 64 + 1)
            g2i = iota4 + (g * 64 + 2)
            g3i = iota4 + (g * 64 + 3)
            cc0 = plsc.load_gather(code_st, [g0i >> 7, g0i & 127])
            cc1 = plsc.load_gather(code_st, [g1i >> 7, g1i & 127])
            cc2 = plsc.load_gather(code_st, [g2i >> 7, g2i & 127])
            cc3 = plsc.load_gather(code_st, [g3i >> 7, g3i & 127])
            f0 = jnp.zeros((16,), jnp.float32)
            f1 = jnp.zeros((16,), jnp.float32)
            for c in range(8):
                wc = wc_v[pl.ds(c * _CH + g * 16, 16)]
                coef = (wc * cc0, wc * cc1, wc * cc2, wc * cc3)
                d0 = c * _CH + g * 16 + iota
                for j in range(8):
                    d2 = jnp.full((16,), j, jnp.int32)
                    v = plsc.load_gather(rows_v, [d0, d2])
                    if j & 1:
                        f1 = f1 + coef[j >> 1] * v
                    else:
                        f0 = f0 + coef[j >> 1] * v
            rowv = g * 16 + iota
            plsc.store_scatter(
                out_cv, [rowv, jnp.full((16,), 2 * l, jnp.int32)], f0)
            plsc.store_scatter(
                out_cv, [rowv, jnp.full((16,), 2 * l + 1, jnp.int32)], f1)
            return carry
        lax.fori_loop(0, _NG, p2, 0)

    def sub_chunk(s, carry):
        base = wid * _BW + s * _CH
        pltpu.sync_copy(x_hbm.at[pl.ds(wid * 96 + s * 12, 12), :], x_st)
        pltpu.sync_copy(code_hbm.at[pl.ds(wid * 128 + s * 16, 16), :], code_st)

        pass1(0, 0)
        fire(0)

        def pair(i, c):
            l = 2 * i
            pass1(l + 1, 1)
            fire(1)
            drain(0)
            pass2(l, 0)

            @pl.when(l + 2 < _N_LEVELS)
            def _():
                pass1(l + 2, 0)
                fire(0)
            drain(1)
            pass2(l + 1, 1)
            return c
        lax.fori_loop(0, _N_LEVELS // 2, pair, 0)

        pltpu.sync_copy(out_cv, out_hbm.at[pl.ds(base, _CH), :])
        return carry

    lax.fori_loop(0, _NSUB, sub_chunk, 0)


_TPB = 4096
_TPW = _T // _NW
_TPBLKS = _TPW // _TPB


_TPN = _N_LEVELS * _TPBLKS


def _tp_body(t4_hbm, tbl8_hbm, st0, st1, ov0, ov1, si0, si1, so0, so1):
    wid = lax.axis_index("s") * _NC + lax.axis_index("c")
    iota = lax.iota(jnp.int32, 16)
    i_half = iota >> 1
    i_par = iota & 1
    sts = (st0, st1)
    ovs = (ov0, ov1)
    sis = (si0, si1)
    sos = (so0, so1)

    def fire_in(i, slot):
        l = i >> 2
        r64 = (wid * _TPW + (i & 3) * _TPB) // 64
        for h in range(_N_HASH):
            pltpu.make_async_copy(
                t4_hbm.at[h, l, pl.ds(r64, _TPB // 64), :],
                sts[slot].at[h], sis[slot]).start()

    def wait_in(i, slot):
        l = i >> 2
        r64 = (wid * _TPW + (i & 3) * _TPB) // 64
        for h in range(_N_HASH):
            pltpu.make_async_copy(
                t4_hbm.at[h, l, pl.ds(r64, _TPB // 64), :],
                sts[slot].at[h], sis[slot]).wait()

    def out_copy(i, slot):
        l = i >> 2
        r0 = wid * _TPW + (i & 3) * _TPB
        return pltpu.make_async_copy(
            ovs[slot], tbl8_hbm.at[pl.ds(l * _T + r0, _TPB), :], sos[slot])

    def compute(i, slot):
        st = sts[slot]
        ov = ovs[slot]

        def q_loop(q, c3):
            dr = i_half + q * 8
            for h in range(_N_HASH):
                v = st[h, q >> 3, pl.ds((q & 7) * 16, 16)]
                plsc.store_scatter(ov, [dr, i_par + 2 * h], v)
            return c3
        lax.fori_loop(0, _TPB // 8, q_loop, 0)

    fire_in(0, 0)

    def step(j, carry):
        i = 2 * j
        fire_in(i + 1, 1)
        wait_in(i, 0)

        @pl.when(i >= 2)
        def _():
            out_copy(i - 2, 0).wait()
        compute(i, 0)
        out_copy(i, 0).start()

        @pl.when(i + 2 < _TPN)
        def _():
            fire_in(i + 2, 0)
        wait_in(i + 1, 1)

        @pl.when(i >= 1)
        def _():
            out_copy(i - 1, 1).wait()
        compute(i + 1, 1)
        out_copy(i + 1, 1).start()
        return carry
    lax.fori_loop(0, _TPN // 2, step, 0)
    out_copy(_TPN - 2, 0).wait()
    out_copy(_TPN - 1, 1).wait()


_tp_call = functools.partial(
    pl.kernel,
    mesh=plsc.VectorSubcoreMesh(core_axis_name="c", subcore_axis_name="s"),
    compiler_params=pltpu.CompilerParams(
        needs_layout_passes=False, use_tc_tiling_on_sc=False),
    out_type=jax.ShapeDtypeStruct((_N_LEVELS * _T, _N_HASH * 2), jnp.float32),
    scratch_types=[
        pltpu.VMEM((_N_HASH, _TPB // 64, 128), jnp.float32),
        pltpu.VMEM((_N_HASH, _TPB // 64, 128), jnp.float32),
        pltpu.VMEM((_TPB, _N_HASH * 2), jnp.float32),
        pltpu.VMEM((_TPB, _N_HASH * 2), jnp.float32),
        pltpu.SemaphoreType.DMA,
        pltpu.SemaphoreType.DMA,
        pltpu.SemaphoreType.DMA,
        pltpu.SemaphoreType.DMA,
    ],
)(_tp_body)


_sc_call = functools.partial(
    pl.kernel,
    mesh=plsc.VectorSubcoreMesh(core_axis_name="c", subcore_axis_name="s"),
    compiler_params=pltpu.CompilerParams(
        needs_layout_passes=False, use_tc_tiling_on_sc=False),
    out_type=jax.ShapeDtypeStruct((_B, 2 * _N_LEVELS), jnp.float32),
    scratch_types=[
        pltpu.VMEM((12, 128), jnp.float32),
        pltpu.VMEM((16, 128), jnp.float32),
        pltpu.VMEM((_IDX_ROWS, 128), jnp.int32),
        pltpu.VMEM((_IDX_ROWS, 128), jnp.int32),
        pltpu.VMEM((_NIDX, 8), jnp.float32),
        pltpu.VMEM((_NIDX, 8), jnp.float32),
        pltpu.VMEM((_NIDX,), jnp.float32),
        pltpu.VMEM((_NIDX,), jnp.float32),
        pltpu.VMEM((_CH, 2 * _N_LEVELS), jnp.float32),
        pltpu.VMEM((_N_LEVELS,), jnp.float32),
        pltpu.SMEM((_N_LEVELS,), jnp.float32),
        pltpu.SMEM((_N_LEVELS,), jnp.float32),
        pltpu.SemaphoreType.DMA,
        pltpu.SemaphoreType.DMA,
    ],
)(_sc_body)


def kernel(in_tensor, conditioning_code, tables):
    tbl8 = _tp_call(tables.reshape(_N_HASH, _N_LEVELS, _T // 64, 128))
    resf = jnp.asarray([float(r) for r in _RES], jnp.float32)
    stridef = jnp.asarray([float(r + 1) for r in _RES], jnp.float32)
    return _sc_call(tbl8, in_tensor.reshape(_B * 3 // 128, 128),
                    conditioning_code.reshape(_B * 4 // 128, 128),
                    resf, stridef)
